# Initial kernel scaffold; baseline (speedup 1.0000x reference)
import jax
import jax.numpy as jnp
from jax.experimental import pallas as pl


def kernel(x, edge_index, pos_edge_index, neg_edge_index, e, W_msg, b_msg, W_ih, W_hh, b_ih, b_hh, W1, b1, W2, b2):
    raise NotImplementedError("write your pallas kernel here")


# speedup vs baseline: 6.7179x; 6.7179x over previous
import functools

import jax
import jax.numpy as jnp
from jax import lax
from jax.experimental import pallas as pl
from jax.experimental.pallas import tpu as pltpu
from jax.experimental.pallas import tpu_sc as plsc

N = 10000
E = 320000
P = 10000
H = 128

NC = 2
NS = 16
NW = NC * NS
KI = 125
ROWS = E // KI
RPW = ROWS // NW
CPT = 624
CPT_LAST = N - (NS - 1) * CPT
GKI = 128
GPAD = 4 * P + (-(4 * P) % (NW * GKI))
GPW = GPAD // (NW * GKI)

BLK = 1000

_MESH = plsc.VectorSubcoreMesh(
    core_axis_name="c", subcore_axis_name="s", num_cores=NC, num_subcores=NS)


def _scatter_body(m_hbm, src_hbm, dst_hbm, z_hbm, out_hbm,
                  sidx, didx, rows0, acc, sem0):
    c = lax.axis_index("c")
    s = lax.axis_index("s")
    w = s * NC + c

    base = s * CPT
    pltpu.sync_copy(z_hbm.at[pl.ds(0, CPT)], acc.at[pl.ds(base, CPT)])

    @pl.when(s == NS - 1)
    def _zero_tail():
        pltpu.sync_copy(z_hbm.at[pl.ds(0, CPT_LAST - CPT)],
                        acc.at[pl.ds(NS * CPT, CPT_LAST - CPT)])
    plsc.subcore_barrier()

    pltpu.sync_copy(src_hbm.at[pl.ds(w * RPW, RPW)], sidx)
    pltpu.sync_copy(dst_hbm.at[pl.ds(w * RPW, RPW)], didx)

    def _step(j, carry):
        pltpu.async_copy(m_hbm.at[sidx.at[j]], rows0, sem0).wait()
        pltpu.sync_copy(rows0, acc.at[didx.at[j]], add=True)
        return carry
    lax.fori_loop(0, RPW, _step, 0)
    plsc.subcore_barrier()

    pltpu.sync_copy(acc.at[pl.ds(base, CPT)], out_hbm.at[c, pl.ds(base, CPT)])

    @pl.when(s == NS - 1)
    def _out_tail():
        pltpu.sync_copy(acc.at[pl.ds(NS * CPT, CPT_LAST - CPT)],
                        out_hbm.at[c, pl.ds(NS * CPT, CPT_LAST - CPT)])


_sc_scatter = functools.partial(
    pl.kernel,
    out_type=jax.ShapeDtypeStruct((NC, N, H), jnp.float32),
    mesh=_MESH,
    scratch_types=[
        pltpu.VMEM((RPW, KI), jnp.int32),
        pltpu.VMEM((RPW, KI), jnp.int32),
        pltpu.VMEM((KI, H), jnp.float32),
        pltpu.VMEM_SHARED((N, H), jnp.float32),
        pltpu.SemaphoreType.DMA,
    ],
)(_scatter_body)


def _gather_body(h_hbm, idx_hbm, out_hbm, gidx, rows0, rows1, sem0, sem1):
    c = lax.axis_index("c")
    s = lax.axis_index("s")
    w = s * NC + c

    pltpu.sync_copy(idx_hbm.at[w], gidx)
    for q in range(GPW // 2):
        j0 = 2 * q
        cp0 = pltpu.async_copy(h_hbm.at[gidx.at[j0]], rows0, sem0)
        cp1 = pltpu.async_copy(h_hbm.at[gidx.at[j0 + 1]], rows1, sem1)
        cp0.wait()
        pltpu.sync_copy(rows0, out_hbm.at[pl.ds((w * GPW + j0) * GKI, GKI)])
        cp1.wait()
        pltpu.sync_copy(rows1, out_hbm.at[pl.ds((w * GPW + j0 + 1) * GKI, GKI)])


_sc_gather = functools.partial(
    pl.kernel,
    out_type=jax.ShapeDtypeStruct((GPAD, H), jnp.float32),
    mesh=_MESH,
    scratch_types=[
        pltpu.VMEM((GPW, GKI), jnp.int32),
        pltpu.VMEM((GKI, H), jnp.float32),
        pltpu.VMEM((GKI, H), jnp.float32),
        pltpu.SemaphoreType.DMA,
        pltpu.SemaphoreType.DMA,
    ],
)(_gather_body)


def _msg_body(h_ref, wT_ref, b_ref, o_ref):
    o_ref[...] = (jnp.dot(h_ref[...], wT_ref[...],
                          preferred_element_type=jnp.float32) + b_ref[...])


_tc_msg = pl.pallas_call(
    _msg_body,
    grid=(N // BLK,),
    in_specs=[
        pl.BlockSpec((BLK, H), lambda i: (i, 0)),
        pl.BlockSpec((H, H), lambda i: (0, 0)),
        pl.BlockSpec((1, H), lambda i: (0, 0)),
    ],
    out_specs=pl.BlockSpec((BLK, H), lambda i: (i, 0)),
    out_shape=jax.ShapeDtypeStruct((N, H), jnp.float32),
)


def _gru_body(a_ref, h_ref, wihT_ref, whhT_ref, bih_ref, bhh_ref,
              wmT_ref, bm_ref, oh_ref, om_ref):
    h = h_ref[...]
    a = a_ref[0] + a_ref[1]
    gi = jnp.dot(a, wihT_ref[...], preferred_element_type=jnp.float32) + bih_ref[...]
    gh = jnp.dot(h, whhT_ref[...], preferred_element_type=jnp.float32) + bhh_ref[...]
    r = jax.nn.sigmoid(gi[:, :H] + gh[:, :H])
    z = jax.nn.sigmoid(gi[:, H:2 * H] + gh[:, H:2 * H])
    n = jnp.tanh(gi[:, 2 * H:] + r * gh[:, 2 * H:])
    hn = (1.0 - z) * n + z * h
    oh_ref[...] = hn
    om_ref[...] = (jnp.dot(hn, wmT_ref[...],
                           preferred_element_type=jnp.float32) + bm_ref[...])


_tc_gru = pl.pallas_call(
    _gru_body,
    grid=(N // BLK,),
    in_specs=[
        pl.BlockSpec((NC, BLK, H), lambda i: (0, i, 0)),
        pl.BlockSpec((BLK, H), lambda i: (i, 0)),
        pl.BlockSpec((H, 3 * H), lambda i: (0, 0)),
        pl.BlockSpec((H, 3 * H), lambda i: (0, 0)),
        pl.BlockSpec((1, 3 * H), lambda i: (0, 0)),
        pl.BlockSpec((1, 3 * H), lambda i: (0, 0)),
        pl.BlockSpec((H, H), lambda i: (0, 0)),
        pl.BlockSpec((1, H), lambda i: (0, 0)),
    ],
    out_specs=[
        pl.BlockSpec((BLK, H), lambda i: (i, 0)),
        pl.BlockSpec((BLK, H), lambda i: (i, 0)),
    ],
    out_shape=[
        jax.ShapeDtypeStruct((N, H), jnp.float32),
        jax.ShapeDtypeStruct((N, H), jnp.float32),
    ],
)


def _pred_body(pa_ref, pb_ref, na_ref, nb_ref, w1T_ref, b1_ref, w2T_ref,
               b2_ref, op_ref, on_ref):
    def head(u):
        v = jnp.dot(u, w1T_ref[...], preferred_element_type=jnp.float32) + b1_ref[...]
        v = jnp.where(v > 0, v, 0.2 * v)
        return jnp.dot(v, w2T_ref[...], preferred_element_type=jnp.float32) + b2_ref[...]

    op_ref[...] = head(pa_ref[...] * pb_ref[...])
    on_ref[...] = head(na_ref[...] * nb_ref[...])


_tc_pred = pl.pallas_call(
    _pred_body,
    grid=(P // BLK,),
    in_specs=[
        pl.BlockSpec((BLK, H), lambda i: (i, 0)),
        pl.BlockSpec((BLK, H), lambda i: (i, 0)),
        pl.BlockSpec((BLK, H), lambda i: (i, 0)),
        pl.BlockSpec((BLK, H), lambda i: (i, 0)),
        pl.BlockSpec((H, H // 2), lambda i: (0, 0)),
        pl.BlockSpec((1, H // 2), lambda i: (0, 0)),
        pl.BlockSpec((H // 2, 1), lambda i: (0, 0)),
        pl.BlockSpec((1, 1), lambda i: (0, 0)),
    ],
    out_specs=[
        pl.BlockSpec((BLK, 1), lambda i: (i, 0)),
        pl.BlockSpec((BLK, 1), lambda i: (i, 0)),
    ],
    out_shape=[
        jax.ShapeDtypeStruct((P, 1), jnp.float32),
        jax.ShapeDtypeStruct((P, 1), jnp.float32),
    ],
)


def kernel(x, edge_index, pos_edge_index, neg_edge_index, e,
           W_msg, b_msg, W_ih, W_hh, b_ih, b_hh, W1, b1, W2, b2):
    src_r = edge_index[0].reshape(ROWS, KI)
    dst_r = edge_index[1].reshape(ROWS, KI)
    wmT = W_msg.T
    bm = b_msg.reshape(1, H)

    h = x
    m = _tc_msg(h, wmT, bm)

    zrows = jnp.zeros((CPT, H), jnp.float32)

    def _step(_, hm):
        h_t, m_t = hm
        a = _sc_scatter(m_t, src_r, dst_r, zrows)
        return _tc_gru(a, h_t,
                       W_ih.T, W_hh.T, b_ih.reshape(1, -1), b_hh.reshape(1, -1),
                       wmT, bm)

    h, m = lax.fori_loop(0, 3, _step, (h, m))

    idx_all = jnp.concatenate(
        [pos_edge_index[0], pos_edge_index[1],
         neg_edge_index[0], neg_edge_index[1],
         jnp.zeros((GPAD - 4 * P,), jnp.int32)]).reshape(NW, GPW, GKI)
    rows = _sc_gather(h, idx_all)
    h_pos, h_neg = _tc_pred(rows[0:P], rows[P:2 * P], rows[2 * P:3 * P],
                            rows[3 * P:4 * P],
                            W1.T, b1.reshape(1, -1), W2.T, b2.reshape(1, 1))
    return (h_pos, h_neg, h)


# speedup vs baseline: 7.4203x; 1.1046x over previous
import functools

import jax
import jax.numpy as jnp
from jax import lax
from jax.experimental import pallas as pl
from jax.experimental.pallas import tpu as pltpu
from jax.experimental.pallas import tpu_sc as plsc

N = 10000
E = 320000
P = 10000
H = 128

NC = 2
NS = 16
NW = NC * NS
KI = 125
ROWS = E // KI
RPW = ROWS // NW
CPT = 624
CPT_LAST = N - (NS - 1) * CPT
GKI = 128
GPAD = 4 * P + (-(4 * P) % (NW * GKI))
GPW = GPAD // (NW * GKI)

BLK = 1000

_MESH = plsc.VectorSubcoreMesh(
    core_axis_name="c", subcore_axis_name="s", num_cores=NC, num_subcores=NS)


def _scatter_body(m_hbm, src_hbm, dst_hbm, z_hbm, out_hbm,
                  sidx, didx, rows0, rows1, acc, sem0, sem1):
    c = lax.axis_index("c")
    s = lax.axis_index("s")
    w = s * NC + c

    base = s * CPT
    pltpu.sync_copy(z_hbm.at[pl.ds(0, CPT)], acc.at[pl.ds(base, CPT)])

    @pl.when(s == NS - 1)
    def _zero_tail():
        pltpu.sync_copy(z_hbm.at[pl.ds(0, CPT_LAST - CPT)],
                        acc.at[pl.ds(NS * CPT, CPT_LAST - CPT)])
    plsc.subcore_barrier()

    HRP = RPW // 2
    for ph in range(2):
        pltpu.sync_copy(src_hbm.at[pl.ds(w * RPW + ph * HRP, HRP)], sidx)
        pltpu.sync_copy(dst_hbm.at[pl.ds(w * RPW + ph * HRP, HRP)], didx)

        def _step(j, carry):
            j0 = 2 * j
            cp0 = pltpu.async_copy(m_hbm.at[sidx.at[j0]], rows0, sem0)
            cp1 = pltpu.async_copy(m_hbm.at[sidx.at[j0 + 1]], rows1, sem1)
            cp0.wait()
            pltpu.sync_copy(rows0, acc.at[didx.at[j0]], add=True)
            cp1.wait()
            pltpu.sync_copy(rows1, acc.at[didx.at[j0 + 1]], add=True)
            return carry
        lax.fori_loop(0, HRP // 2, _step, 0)
    plsc.subcore_barrier()

    pltpu.sync_copy(acc.at[pl.ds(base, CPT)], out_hbm.at[c, pl.ds(base, CPT)])

    @pl.when(s == NS - 1)
    def _out_tail():
        pltpu.sync_copy(acc.at[pl.ds(NS * CPT, CPT_LAST - CPT)],
                        out_hbm.at[c, pl.ds(NS * CPT, CPT_LAST - CPT)])


_sc_scatter = functools.partial(
    pl.kernel,
    out_type=jax.ShapeDtypeStruct((NC, N, H), jnp.float32),
    mesh=_MESH,
    scratch_types=[
        pltpu.VMEM((RPW // 2, KI), jnp.int32),
        pltpu.VMEM((RPW // 2, KI), jnp.int32),
        pltpu.VMEM((KI, H), jnp.float32),
        pltpu.VMEM((KI, H), jnp.float32),
        pltpu.VMEM_SHARED((N, H), jnp.float32),
        pltpu.SemaphoreType.DMA,
        pltpu.SemaphoreType.DMA,
    ],
)(_scatter_body)


def _gather_body(h_hbm, idx_hbm, out_hbm, gidx, rows0, rows1, sem0, sem1):
    c = lax.axis_index("c")
    s = lax.axis_index("s")
    w = s * NC + c

    pltpu.sync_copy(idx_hbm.at[w], gidx)
    for q in range(GPW // 2):
        j0 = 2 * q
        cp0 = pltpu.async_copy(h_hbm.at[gidx.at[j0]], rows0, sem0)
        cp1 = pltpu.async_copy(h_hbm.at[gidx.at[j0 + 1]], rows1, sem1)
        cp0.wait()
        pltpu.sync_copy(rows0, out_hbm.at[pl.ds((w * GPW + j0) * GKI, GKI)])
        cp1.wait()
        pltpu.sync_copy(rows1, out_hbm.at[pl.ds((w * GPW + j0 + 1) * GKI, GKI)])


_sc_gather = functools.partial(
    pl.kernel,
    out_type=jax.ShapeDtypeStruct((GPAD, H), jnp.float32),
    mesh=_MESH,
    scratch_types=[
        pltpu.VMEM((GPW, GKI), jnp.int32),
        pltpu.VMEM((GKI, H), jnp.float32),
        pltpu.VMEM((GKI, H), jnp.float32),
        pltpu.SemaphoreType.DMA,
        pltpu.SemaphoreType.DMA,
    ],
)(_gather_body)


def _msg_body(h_ref, wT_ref, b_ref, o_ref):
    o_ref[...] = (jnp.dot(h_ref[...], wT_ref[...],
                          preferred_element_type=jnp.float32) + b_ref[...])


_tc_msg = pl.pallas_call(
    _msg_body,
    grid=(N // BLK,),
    in_specs=[
        pl.BlockSpec((BLK, H), lambda i: (i, 0)),
        pl.BlockSpec((H, H), lambda i: (0, 0)),
        pl.BlockSpec((1, H), lambda i: (0, 0)),
    ],
    out_specs=pl.BlockSpec((BLK, H), lambda i: (i, 0)),
    out_shape=jax.ShapeDtypeStruct((N, H), jnp.float32),
)


def _gru_body(a_ref, h_ref, wihT_ref, whhT_ref, bih_ref, bhh_ref,
              wmT_ref, bm_ref, oh_ref, om_ref):
    h = h_ref[...]
    a = a_ref[0] + a_ref[1]
    gi = jnp.dot(a, wihT_ref[...], preferred_element_type=jnp.float32) + bih_ref[...]
    gh = jnp.dot(h, whhT_ref[...], preferred_element_type=jnp.float32) + bhh_ref[...]
    r = jax.nn.sigmoid(gi[:, :H] + gh[:, :H])
    z = jax.nn.sigmoid(gi[:, H:2 * H] + gh[:, H:2 * H])
    n = jnp.tanh(gi[:, 2 * H:] + r * gh[:, 2 * H:])
    hn = (1.0 - z) * n + z * h
    oh_ref[...] = hn
    om_ref[...] = (jnp.dot(hn, wmT_ref[...],
                           preferred_element_type=jnp.float32) + bm_ref[...])


_tc_gru = pl.pallas_call(
    _gru_body,
    grid=(N // BLK,),
    in_specs=[
        pl.BlockSpec((NC, BLK, H), lambda i: (0, i, 0)),
        pl.BlockSpec((BLK, H), lambda i: (i, 0)),
        pl.BlockSpec((H, 3 * H), lambda i: (0, 0)),
        pl.BlockSpec((H, 3 * H), lambda i: (0, 0)),
        pl.BlockSpec((1, 3 * H), lambda i: (0, 0)),
        pl.BlockSpec((1, 3 * H), lambda i: (0, 0)),
        pl.BlockSpec((H, H), lambda i: (0, 0)),
        pl.BlockSpec((1, H), lambda i: (0, 0)),
    ],
    out_specs=[
        pl.BlockSpec((BLK, H), lambda i: (i, 0)),
        pl.BlockSpec((BLK, H), lambda i: (i, 0)),
    ],
    out_shape=[
        jax.ShapeDtypeStruct((N, H), jnp.float32),
        jax.ShapeDtypeStruct((N, H), jnp.float32),
    ],
)


def _pred_body(pa_ref, pb_ref, na_ref, nb_ref, w1T_ref, b1_ref, w2T_ref,
               b2_ref, op_ref, on_ref):
    def head(u):
        v = jnp.dot(u, w1T_ref[...], preferred_element_type=jnp.float32) + b1_ref[...]
        v = jnp.where(v > 0, v, 0.2 * v)
        return jnp.dot(v, w2T_ref[...], preferred_element_type=jnp.float32) + b2_ref[...]

    op_ref[...] = head(pa_ref[...] * pb_ref[...])
    on_ref[...] = head(na_ref[...] * nb_ref[...])


_tc_pred = pl.pallas_call(
    _pred_body,
    grid=(P // BLK,),
    in_specs=[
        pl.BlockSpec((BLK, H), lambda i: (i, 0)),
        pl.BlockSpec((BLK, H), lambda i: (i, 0)),
        pl.BlockSpec((BLK, H), lambda i: (i, 0)),
        pl.BlockSpec((BLK, H), lambda i: (i, 0)),
        pl.BlockSpec((H, H // 2), lambda i: (0, 0)),
        pl.BlockSpec((1, H // 2), lambda i: (0, 0)),
        pl.BlockSpec((H // 2, 1), lambda i: (0, 0)),
        pl.BlockSpec((1, 1), lambda i: (0, 0)),
    ],
    out_specs=[
        pl.BlockSpec((BLK, 1), lambda i: (i, 0)),
        pl.BlockSpec((BLK, 1), lambda i: (i, 0)),
    ],
    out_shape=[
        jax.ShapeDtypeStruct((P, 1), jnp.float32),
        jax.ShapeDtypeStruct((P, 1), jnp.float32),
    ],
)


def kernel(x, edge_index, pos_edge_index, neg_edge_index, e,
           W_msg, b_msg, W_ih, W_hh, b_ih, b_hh, W1, b1, W2, b2):
    src_r = edge_index[0].reshape(ROWS, KI)
    dst_r = edge_index[1].reshape(ROWS, KI)
    wmT = W_msg.T
    bm = b_msg.reshape(1, H)

    h = x
    m = _tc_msg(h, wmT, bm)

    zrows = jnp.zeros((CPT, H), jnp.float32)

    def _step(_, hm):
        h_t, m_t = hm
        a = _sc_scatter(m_t, src_r, dst_r, zrows)
        return _tc_gru(a, h_t,
                       W_ih.T, W_hh.T, b_ih.reshape(1, -1), b_hh.reshape(1, -1),
                       wmT, bm)

    h, m = lax.fori_loop(0, 3, _step, (h, m))

    idx_all = jnp.concatenate(
        [pos_edge_index[0], pos_edge_index[1],
         neg_edge_index[0], neg_edge_index[1],
         jnp.zeros((GPAD - 4 * P,), jnp.int32)]).reshape(NW, GPW, GKI)
    rows = _sc_gather(h, idx_all)
    h_pos, h_neg = _tc_pred(rows[0:P], rows[P:2 * P], rows[2 * P:3 * P],
                            rows[3 * P:4 * P],
                            W1.T, b1.reshape(1, -1), W2.T, b2.reshape(1, 1))
    return (h_pos, h_neg, h)


# speedup vs baseline: 8.8322x; 1.1903x over previous
import functools

import jax
import jax.numpy as jnp
from jax import lax
from jax.experimental import pallas as pl
from jax.experimental.pallas import tpu as pltpu
from jax.experimental.pallas import tpu_sc as plsc

N = 10000
E = 320000
P = 10000
H = 128

NC = 2
NS = 16
NW = NC * NS
KI = 125
ROWS = E // KI
RPW = ROWS // NW
CPT = 624
CPT_LAST = N - (NS - 1) * CPT
GKI = 128
GPAD = 4 * P + (-(4 * P) % (NW * GKI))
GPW = GPAD // (NW * GKI)

BLK = 1000

_MESH = plsc.VectorSubcoreMesh(
    core_axis_name="c", subcore_axis_name="s", num_cores=NC, num_subcores=NS)


def _scatter_body(m_hbm, src_hbm, dst_hbm, z_hbm, out_hbm,
                  sidx, didx, rows0, rows1, acc, sem0, sem1):
    c = lax.axis_index("c")
    s = lax.axis_index("s")
    w = s * NC + c

    base = s * CPT
    pltpu.sync_copy(z_hbm.at[pl.ds(0, CPT)], acc.at[pl.ds(base, CPT)])

    @pl.when(s == NS - 1)
    def _zero_tail():
        pltpu.sync_copy(z_hbm.at[pl.ds(0, CPT_LAST - CPT)],
                        acc.at[pl.ds(NS * CPT, CPT_LAST - CPT)])
    plsc.subcore_barrier()

    HRP = RPW // 2
    HPAIR = HRP // 2
    for ph in range(2):
        pltpu.sync_copy(src_hbm.at[pl.ds(w * RPW + ph * HRP, HRP)], sidx)
        pltpu.sync_copy(dst_hbm.at[pl.ds(w * RPW + ph * HRP, HRP)], didx)
        pltpu.async_copy(m_hbm.at[sidx.at[0]], rows0, sem0)

        def _step(p, carry):
            j0 = 2 * p
            pltpu.make_async_copy(m_hbm.at[sidx.at[j0]], rows0, sem0).wait()
            pltpu.async_copy(m_hbm.at[sidx.at[j0 + 1]], rows1, sem1)
            pltpu.sync_copy(rows0, acc.at[didx.at[j0]], add=True)

            @pl.when(p < HPAIR - 1)
            def _prefetch():
                pltpu.async_copy(m_hbm.at[sidx.at[j0 + 2]], rows0, sem0)

            pltpu.make_async_copy(m_hbm.at[sidx.at[j0 + 1]], rows1, sem1).wait()
            pltpu.sync_copy(rows1, acc.at[didx.at[j0 + 1]], add=True)
            return carry
        lax.fori_loop(0, HPAIR, _step, 0)
    plsc.subcore_barrier()

    pltpu.sync_copy(acc.at[pl.ds(base, CPT)], out_hbm.at[c, pl.ds(base, CPT)])

    @pl.when(s == NS - 1)
    def _out_tail():
        pltpu.sync_copy(acc.at[pl.ds(NS * CPT, CPT_LAST - CPT)],
                        out_hbm.at[c, pl.ds(NS * CPT, CPT_LAST - CPT)])


_sc_scatter = functools.partial(
    pl.kernel,
    out_type=jax.ShapeDtypeStruct((NC, N, H), jnp.float32),
    mesh=_MESH,
    scratch_types=[
        pltpu.VMEM((RPW // 2, KI), jnp.int32),
        pltpu.VMEM((RPW // 2, KI), jnp.int32),
        pltpu.VMEM((KI, H), jnp.float32),
        pltpu.VMEM((KI, H), jnp.float32),
        pltpu.VMEM_SHARED((N, H), jnp.float32),
        pltpu.SemaphoreType.DMA,
        pltpu.SemaphoreType.DMA,
    ],
)(_scatter_body)


def _gather_body(h_hbm, idx_hbm, out_hbm, gidx, rows0, rows1, sem0, sem1):
    c = lax.axis_index("c")
    s = lax.axis_index("s")
    w = s * NC + c

    pltpu.sync_copy(idx_hbm.at[w], gidx)
    for q in range(GPW // 2):
        j0 = 2 * q
        cp0 = pltpu.async_copy(h_hbm.at[gidx.at[j0]], rows0, sem0)
        cp1 = pltpu.async_copy(h_hbm.at[gidx.at[j0 + 1]], rows1, sem1)
        cp0.wait()
        pltpu.sync_copy(rows0, out_hbm.at[pl.ds((w * GPW + j0) * GKI, GKI)])
        cp1.wait()
        pltpu.sync_copy(rows1, out_hbm.at[pl.ds((w * GPW + j0 + 1) * GKI, GKI)])


_sc_gather = functools.partial(
    pl.kernel,
    out_type=jax.ShapeDtypeStruct((GPAD, H), jnp.float32),
    mesh=_MESH,
    scratch_types=[
        pltpu.VMEM((GPW, GKI), jnp.int32),
        pltpu.VMEM((GKI, H), jnp.float32),
        pltpu.VMEM((GKI, H), jnp.float32),
        pltpu.SemaphoreType.DMA,
        pltpu.SemaphoreType.DMA,
    ],
)(_gather_body)


def _msg_body(h_ref, wT_ref, b_ref, o_ref):
    o_ref[...] = (jnp.dot(h_ref[...], wT_ref[...],
                          preferred_element_type=jnp.float32) + b_ref[...])


_tc_msg = pl.pallas_call(
    _msg_body,
    grid=(N // BLK,),
    in_specs=[
        pl.BlockSpec((BLK, H), lambda i: (i, 0)),
        pl.BlockSpec((H, H), lambda i: (0, 0)),
        pl.BlockSpec((1, H), lambda i: (0, 0)),
    ],
    out_specs=pl.BlockSpec((BLK, H), lambda i: (i, 0)),
    out_shape=jax.ShapeDtypeStruct((N, H), jnp.float32),
)


def _gru_body(a_ref, h_ref, wihT_ref, whhT_ref, bih_ref, bhh_ref,
              wmT_ref, bm_ref, oh_ref, om_ref):
    h = h_ref[...]
    a = a_ref[0] + a_ref[1]
    gi = jnp.dot(a, wihT_ref[...], preferred_element_type=jnp.float32) + bih_ref[...]
    gh = jnp.dot(h, whhT_ref[...], preferred_element_type=jnp.float32) + bhh_ref[...]
    r = jax.nn.sigmoid(gi[:, :H] + gh[:, :H])
    z = jax.nn.sigmoid(gi[:, H:2 * H] + gh[:, H:2 * H])
    n = jnp.tanh(gi[:, 2 * H:] + r * gh[:, 2 * H:])
    hn = (1.0 - z) * n + z * h
    oh_ref[...] = hn
    om_ref[...] = (jnp.dot(hn, wmT_ref[...],
                           preferred_element_type=jnp.float32) + bm_ref[...])


_tc_gru = pl.pallas_call(
    _gru_body,
    grid=(N // BLK,),
    in_specs=[
        pl.BlockSpec((NC, BLK, H), lambda i: (0, i, 0)),
        pl.BlockSpec((BLK, H), lambda i: (i, 0)),
        pl.BlockSpec((H, 3 * H), lambda i: (0, 0)),
        pl.BlockSpec((H, 3 * H), lambda i: (0, 0)),
        pl.BlockSpec((1, 3 * H), lambda i: (0, 0)),
        pl.BlockSpec((1, 3 * H), lambda i: (0, 0)),
        pl.BlockSpec((H, H), lambda i: (0, 0)),
        pl.BlockSpec((1, H), lambda i: (0, 0)),
    ],
    out_specs=[
        pl.BlockSpec((BLK, H), lambda i: (i, 0)),
        pl.BlockSpec((BLK, H), lambda i: (i, 0)),
    ],
    out_shape=[
        jax.ShapeDtypeStruct((N, H), jnp.float32),
        jax.ShapeDtypeStruct((N, H), jnp.float32),
    ],
)


def _pred_body(pa_ref, pb_ref, na_ref, nb_ref, w1T_ref, b1_ref, w2T_ref,
               b2_ref, op_ref, on_ref):
    def head(u):
        v = jnp.dot(u, w1T_ref[...], preferred_element_type=jnp.float32) + b1_ref[...]
        v = jnp.where(v > 0, v, 0.2 * v)
        return jnp.dot(v, w2T_ref[...], preferred_element_type=jnp.float32) + b2_ref[...]

    op_ref[...] = head(pa_ref[...] * pb_ref[...])
    on_ref[...] = head(na_ref[...] * nb_ref[...])


_tc_pred = pl.pallas_call(
    _pred_body,
    grid=(P // BLK,),
    in_specs=[
        pl.BlockSpec((BLK, H), lambda i: (i, 0)),
        pl.BlockSpec((BLK, H), lambda i: (i, 0)),
        pl.BlockSpec((BLK, H), lambda i: (i, 0)),
        pl.BlockSpec((BLK, H), lambda i: (i, 0)),
        pl.BlockSpec((H, H // 2), lambda i: (0, 0)),
        pl.BlockSpec((1, H // 2), lambda i: (0, 0)),
        pl.BlockSpec((H // 2, 1), lambda i: (0, 0)),
        pl.BlockSpec((1, 1), lambda i: (0, 0)),
    ],
    out_specs=[
        pl.BlockSpec((BLK, 1), lambda i: (i, 0)),
        pl.BlockSpec((BLK, 1), lambda i: (i, 0)),
    ],
    out_shape=[
        jax.ShapeDtypeStruct((P, 1), jnp.float32),
        jax.ShapeDtypeStruct((P, 1), jnp.float32),
    ],
)


def kernel(x, edge_index, pos_edge_index, neg_edge_index, e,
           W_msg, b_msg, W_ih, W_hh, b_ih, b_hh, W1, b1, W2, b2):
    src_r = edge_index[0].reshape(ROWS, KI)
    dst_r = edge_index[1].reshape(ROWS, KI)
    wmT = W_msg.T
    bm = b_msg.reshape(1, H)

    h = x
    m = _tc_msg(h, wmT, bm)

    zrows = jnp.zeros((CPT, H), jnp.float32)

    def _step(_, hm):
        h_t, m_t = hm
        a = _sc_scatter(m_t, src_r, dst_r, zrows)
        return _tc_gru(a, h_t,
                       W_ih.T, W_hh.T, b_ih.reshape(1, -1), b_hh.reshape(1, -1),
                       wmT, bm)

    h, m = lax.fori_loop(0, 3, _step, (h, m))

    idx_all = jnp.concatenate(
        [pos_edge_index[0], pos_edge_index[1],
         neg_edge_index[0], neg_edge_index[1],
         jnp.zeros((GPAD - 4 * P,), jnp.int32)]).reshape(NW, GPW, GKI)
    rows = _sc_gather(h, idx_all)
    h_pos, h_neg = _tc_pred(rows[0:P], rows[P:2 * P], rows[2 * P:3 * P],
                            rows[3 * P:4 * P],
                            W1.T, b1.reshape(1, -1), W2.T, b2.reshape(1, 1))
    return (h_pos, h_neg, h)


# speedup vs baseline: 8.9646x; 1.0150x over previous
import functools

import jax
import jax.numpy as jnp
from jax import lax
from jax.experimental import pallas as pl
from jax.experimental.pallas import tpu as pltpu
from jax.experimental.pallas import tpu_sc as plsc

N = 10000
E = 320000
P = 10000
H = 128

NC = 2
NS = 16
NW = NC * NS
KI = 125
ROWS = E // KI
RPW = ROWS // NW
CPT = 624
CPT_LAST = N - (NS - 1) * CPT
GKI = 128
GPAD = 4 * P + (-(4 * P) % (NW * GKI))
GPW = GPAD // (NW * GKI)

BLK = 1000

_MESH = plsc.VectorSubcoreMesh(
    core_axis_name="c", subcore_axis_name="s", num_cores=NC, num_subcores=NS)


def _scatter_body(m_hbm, src_hbm, dst_hbm, z_hbm, out_hbm,
                  sidx, didx, rows0, rows1, acc, sem0, sem1):
    c = lax.axis_index("c")
    s = lax.axis_index("s")
    w = s * NC + c

    base = s * CPT
    pltpu.sync_copy(z_hbm.at[pl.ds(0, CPT)], acc.at[pl.ds(base, CPT)])

    @pl.when(s == NS - 1)
    def _zero_tail():
        pltpu.sync_copy(z_hbm.at[pl.ds(0, CPT_LAST - CPT)],
                        acc.at[pl.ds(NS * CPT, CPT_LAST - CPT)])
    plsc.subcore_barrier()

    HRP = RPW // 2
    HPAIR = HRP // 2
    for ph in range(2):
        pltpu.sync_copy(src_hbm.at[pl.ds(w * RPW + ph * HRP, HRP)], sidx)
        pltpu.sync_copy(dst_hbm.at[pl.ds(w * RPW + ph * HRP, HRP)], didx)
        pltpu.async_copy(m_hbm.at[sidx.at[0]], rows0, sem0)

        def _step(p, carry):
            j0 = 2 * p
            pltpu.make_async_copy(m_hbm.at[sidx.at[j0]], rows0, sem0).wait()
            pltpu.async_copy(m_hbm.at[sidx.at[j0 + 1]], rows1, sem1)
            pltpu.sync_copy(rows0, acc.at[didx.at[j0]], add=True)

            @pl.when(p < HPAIR - 1)
            def _prefetch():
                pltpu.async_copy(m_hbm.at[sidx.at[j0 + 2]], rows0, sem0)

            pltpu.make_async_copy(m_hbm.at[sidx.at[j0 + 1]], rows1, sem1).wait()
            pltpu.sync_copy(rows1, acc.at[didx.at[j0 + 1]], add=True)
            return carry
        lax.fori_loop(0, HPAIR, _step, 0)
    plsc.subcore_barrier()

    pltpu.sync_copy(acc.at[pl.ds(base, CPT)], out_hbm.at[c, pl.ds(base, CPT)])

    @pl.when(s == NS - 1)
    def _out_tail():
        pltpu.sync_copy(acc.at[pl.ds(NS * CPT, CPT_LAST - CPT)],
                        out_hbm.at[c, pl.ds(NS * CPT, CPT_LAST - CPT)])


_sc_scatter = functools.partial(
    pl.kernel,
    out_type=jax.ShapeDtypeStruct((NC, N, H), jnp.float32),
    mesh=_MESH,
    scratch_types=[
        pltpu.VMEM((RPW // 2, KI), jnp.int32),
        pltpu.VMEM((RPW // 2, KI), jnp.int32),
        pltpu.VMEM((KI, H), jnp.float32),
        pltpu.VMEM((KI, H), jnp.float32),
        pltpu.VMEM_SHARED((N, H), jnp.float32),
        pltpu.SemaphoreType.DMA,
        pltpu.SemaphoreType.DMA,
    ],
)(_scatter_body)


def _gather_body(h_hbm, idx_hbm, out_hbm, gidx, rows0, rows1, sem0, sem1):
    c = lax.axis_index("c")
    s = lax.axis_index("s")
    w = s * NC + c

    pltpu.sync_copy(idx_hbm.at[w], gidx)
    cp = {0: pltpu.async_copy(h_hbm.at[gidx.at[0]], rows0, sem0)}
    for q in range(GPW // 2):
        j0 = 2 * q
        cp.pop(j0).wait()
        cp[j0 + 1] = pltpu.async_copy(h_hbm.at[gidx.at[j0 + 1]], rows1, sem1)
        pltpu.sync_copy(rows0, out_hbm.at[pl.ds((w * GPW + j0) * GKI, GKI)])
        if j0 + 2 < GPW:
            cp[j0 + 2] = pltpu.async_copy(h_hbm.at[gidx.at[j0 + 2]], rows0, sem0)
        cp.pop(j0 + 1).wait()
        pltpu.sync_copy(rows1, out_hbm.at[pl.ds((w * GPW + j0 + 1) * GKI, GKI)])


_sc_gather = functools.partial(
    pl.kernel,
    out_type=jax.ShapeDtypeStruct((GPAD, H), jnp.float32),
    mesh=_MESH,
    scratch_types=[
        pltpu.VMEM((GPW, GKI), jnp.int32),
        pltpu.VMEM((GKI, H), jnp.float32),
        pltpu.VMEM((GKI, H), jnp.float32),
        pltpu.SemaphoreType.DMA,
        pltpu.SemaphoreType.DMA,
    ],
)(_gather_body)


def _msg_body(h_ref, wT_ref, b_ref, o_ref):
    o_ref[...] = (jnp.dot(h_ref[...], wT_ref[...],
                          preferred_element_type=jnp.float32) + b_ref[...])


_tc_msg = pl.pallas_call(
    _msg_body,
    grid=(N // BLK,),
    in_specs=[
        pl.BlockSpec((BLK, H), lambda i: (i, 0)),
        pl.BlockSpec((H, H), lambda i: (0, 0)),
        pl.BlockSpec((1, H), lambda i: (0, 0)),
    ],
    out_specs=pl.BlockSpec((BLK, H), lambda i: (i, 0)),
    out_shape=jax.ShapeDtypeStruct((N, H), jnp.float32),
)


def _gru_body(a_ref, h_ref, wihT_ref, whhT_ref, bih_ref, bhh_ref,
              wmT_ref, bm_ref, oh_ref, om_ref):
    h = h_ref[...]
    a = a_ref[0] + a_ref[1]
    gi = jnp.dot(a, wihT_ref[...], preferred_element_type=jnp.float32) + bih_ref[...]
    gh = jnp.dot(h, whhT_ref[...], preferred_element_type=jnp.float32) + bhh_ref[...]
    r = jax.nn.sigmoid(gi[:, :H] + gh[:, :H])
    z = jax.nn.sigmoid(gi[:, H:2 * H] + gh[:, H:2 * H])
    n = jnp.tanh(gi[:, 2 * H:] + r * gh[:, 2 * H:])
    hn = (1.0 - z) * n + z * h
    oh_ref[...] = hn
    om_ref[...] = (jnp.dot(hn, wmT_ref[...],
                           preferred_element_type=jnp.float32) + bm_ref[...])


_tc_gru = pl.pallas_call(
    _gru_body,
    grid=(N // BLK,),
    in_specs=[
        pl.BlockSpec((NC, BLK, H), lambda i: (0, i, 0)),
        pl.BlockSpec((BLK, H), lambda i: (i, 0)),
        pl.BlockSpec((H, 3 * H), lambda i: (0, 0)),
        pl.BlockSpec((H, 3 * H), lambda i: (0, 0)),
        pl.BlockSpec((1, 3 * H), lambda i: (0, 0)),
        pl.BlockSpec((1, 3 * H), lambda i: (0, 0)),
        pl.BlockSpec((H, H), lambda i: (0, 0)),
        pl.BlockSpec((1, H), lambda i: (0, 0)),
    ],
    out_specs=[
        pl.BlockSpec((BLK, H), lambda i: (i, 0)),
        pl.BlockSpec((BLK, H), lambda i: (i, 0)),
    ],
    out_shape=[
        jax.ShapeDtypeStruct((N, H), jnp.float32),
        jax.ShapeDtypeStruct((N, H), jnp.float32),
    ],
)


def _pred_body(pa_ref, pb_ref, na_ref, nb_ref, w1T_ref, b1_ref, w2T_ref,
               b2_ref, op_ref, on_ref):
    def head(u):
        v = jnp.dot(u, w1T_ref[...], preferred_element_type=jnp.float32) + b1_ref[...]
        v = jnp.where(v > 0, v, 0.2 * v)
        return jnp.dot(v, w2T_ref[...], preferred_element_type=jnp.float32) + b2_ref[...]

    op_ref[...] = head(pa_ref[...] * pb_ref[...])
    on_ref[...] = head(na_ref[...] * nb_ref[...])


_tc_pred = pl.pallas_call(
    _pred_body,
    grid=(P // BLK,),
    in_specs=[
        pl.BlockSpec((BLK, H), lambda i: (i, 0)),
        pl.BlockSpec((BLK, H), lambda i: (i, 0)),
        pl.BlockSpec((BLK, H), lambda i: (i, 0)),
        pl.BlockSpec((BLK, H), lambda i: (i, 0)),
        pl.BlockSpec((H, H // 2), lambda i: (0, 0)),
        pl.BlockSpec((1, H // 2), lambda i: (0, 0)),
        pl.BlockSpec((H // 2, 1), lambda i: (0, 0)),
        pl.BlockSpec((1, 1), lambda i: (0, 0)),
    ],
    out_specs=[
        pl.BlockSpec((BLK, 1), lambda i: (i, 0)),
        pl.BlockSpec((BLK, 1), lambda i: (i, 0)),
    ],
    out_shape=[
        jax.ShapeDtypeStruct((P, 1), jnp.float32),
        jax.ShapeDtypeStruct((P, 1), jnp.float32),
    ],
)


def kernel(x, edge_index, pos_edge_index, neg_edge_index, e,
           W_msg, b_msg, W_ih, W_hh, b_ih, b_hh, W1, b1, W2, b2):
    src_r = edge_index[0].reshape(ROWS, KI)
    dst_r = edge_index[1].reshape(ROWS, KI)
    wmT = W_msg.T
    bm = b_msg.reshape(1, H)

    h = x
    m = _tc_msg(h, wmT, bm)

    zrows = jnp.zeros((CPT, H), jnp.float32)

    def _step(_, hm):
        h_t, m_t = hm
        a = _sc_scatter(m_t, src_r, dst_r, zrows)
        return _tc_gru(a, h_t,
                       W_ih.T, W_hh.T, b_ih.reshape(1, -1), b_hh.reshape(1, -1),
                       wmT, bm)

    h, m = lax.fori_loop(0, 3, _step, (h, m))

    idx_all = jnp.concatenate(
        [pos_edge_index[0], pos_edge_index[1],
         neg_edge_index[0], neg_edge_index[1],
         jnp.zeros((GPAD - 4 * P,), jnp.int32)]).reshape(NW, GPW, GKI)
    rows = _sc_gather(h, idx_all)
    h_pos, h_neg = _tc_pred(rows[0:P], rows[P:2 * P], rows[2 * P:3 * P],
                            rows[3 * P:4 * P],
                            W1.T, b1.reshape(1, -1), W2.T, b2.reshape(1, 1))
    return (h_pos, h_neg, h)
